# Initial kernel scaffold; baseline (speedup 1.0000x reference)
import jax
import jax.numpy as jnp
from jax.experimental import pallas as pl


def kernel(logits, labels):
    raise NotImplementedError("write your pallas kernel here")


# speedup vs baseline: 16.9359x; 16.9359x over previous
import functools

import jax
import jax.numpy as jnp
from jax import lax
from jax.experimental import pallas as pl
from jax.experimental.pallas import tpu as pltpu
from jax.experimental.pallas import tpu_sc as plsc

P_TOTAL = 8 * 1 * 512 * 512
NB = 16384
NTILES = 32
CHUNK = P_TOTAL // NTILES
W = 2048
ROWS = 2048
COLS = P_TOTAL // ROWS
MM_BLK = 256


def _minmax_body(x_ref, l_ref, mn_ref, mx_ref):
    x = x_ref[...]
    lf = l_ref[...].astype(jnp.float32)
    e = 1.0 - x * (2.0 * lf - 1.0)
    m = jnp.min(e)
    M = jnp.max(e)

    @pl.when(pl.program_id(0) == 0)
    def _():
        mn_ref[...] = jnp.full((1, 1), m)
        mx_ref[...] = jnp.full((1, 1), M)

    mn_ref[...] = jnp.minimum(mn_ref[...], m)
    mx_ref[...] = jnp.maximum(mx_ref[...], M)


def _minmax(x2d, l2d):
    return pl.pallas_call(
        _minmax_body,
        grid=(ROWS // MM_BLK,),
        in_specs=[
            pl.BlockSpec((MM_BLK, COLS), lambda i: (i, 0)),
            pl.BlockSpec((MM_BLK, COLS), lambda i: (i, 0)),
        ],
        out_specs=[
            pl.BlockSpec((1, 1), lambda i: (0, 0)),
            pl.BlockSpec((1, 1), lambda i: (0, 0)),
        ],
        out_shape=[
            jax.ShapeDtypeStruct((1, 1), jnp.float32),
            jax.ShapeDtypeStruct((1, 1), jnp.float32),
        ],
    )(x2d, l2d)


def _sc_hist_body(x_hbm, l_hbm, params_hbm, out_hbm, table, xbuf, lbuf, pbuf):
    wid = lax.axis_index("s") * 2 + lax.axis_index("c")
    base = wid * CHUNK

    pltpu.sync_copy(params_hbm, pbuf)
    pv = pbuf[pl.ds(0, 16)]
    emin = pv[0]
    scale = pv[1]

    def zero_body(i, c):
        table[pl.ds(i * 16, 16)] = jnp.zeros((16,), jnp.float32)
        return c

    lax.fori_loop(0, 4 * NB // 16, zero_body, 0)

    ones = jnp.full((16,), 1.0, jnp.float32)

    def win_body(w, c):
        pltpu.sync_copy(x_hbm.at[pl.ds(base + w * W, W)], xbuf)
        pltpu.sync_copy(l_hbm.at[pl.ds(base + w * W, W)], lbuf)

        def elem_body(j, c2):
            x = xbuf[pl.ds(j * 16, 16)]
            li = lbuf[pl.ds(j * 16, 16)]
            lf = li.astype(jnp.float32)
            e = 1.0 - x * (2.0 * lf - 1.0)
            g = jnp.where(e > 0.0, e + 1.0, jnp.exp(e))
            b = ((e - emin) * scale).astype(jnp.int32)
            b = jnp.clip(b, 0, NB - 1)
            idx = b + li * NB
            plsc.addupdate_scatter(table, [idx], ones)
            plsc.addupdate_scatter(table, [idx + 2 * NB], g)
            return c2

        lax.fori_loop(0, W // 16, elem_body, 0)
        return c

    lax.fori_loop(0, CHUNK // W, win_body, 0)
    pltpu.sync_copy(table, out_hbm.at[wid])


_sc_hist = functools.partial(
    pl.kernel,
    out_type=jax.ShapeDtypeStruct((NTILES, 4 * NB), jnp.float32),
    mesh=plsc.VectorSubcoreMesh(core_axis_name="c", subcore_axis_name="s"),
    compiler_params=pltpu.CompilerParams(needs_layout_passes=False),
    scratch_types=[
        pltpu.VMEM((4 * NB,), jnp.float32),
        pltpu.VMEM((W,), jnp.float32),
        pltpu.VMEM((W,), jnp.int32),
        pltpu.VMEM((16,), jnp.float32),
    ],
)(_sc_hist_body)


def _finish_body(h_ref, out_ref):
    h = h_ref[...]
    s = jnp.sum(h, axis=0)
    cp = s[0]
    cn = s[1]
    gp = s[2]
    gn = s[3]

    rows = lax.broadcasted_iota(jnp.int32, (128, 128), 0)
    cols = lax.broadcasted_iota(jnp.int32, (128, 128), 1)
    upper = (rows <= cols).astype(jnp.float32)
    lstrict = (cols < rows).astype(jnp.float32)

    def incl_cumsum(a):
        within = jnp.dot(a, upper, preferred_element_type=jnp.float32)
        row_tot = jnp.sum(a, axis=1, keepdims=True)
        row_pre = jnp.dot(lstrict, row_tot,
                          preferred_element_type=jnp.float32)
        return within + row_pre

    G = jnp.sum(cp)
    P = G - incl_cumsum(cp)
    N = jnp.sum(cn) - incl_cumsum(cn)
    tn = cn
    tp = cp
    GN = G + N
    term_pos = jnp.where(tp > 0.0, gp / (GN + tn), 0.0)
    grp = jnp.where(GN > 0.0,
                    (G - P) * (1.0 / GN - 1.0 / (GN + tn)),
                    1.0)
    term_neg = jnp.where(tn > 0.0, (gn / tn) * grp, 0.0)
    out_ref[...] = jnp.full((1, 1), jnp.sum(term_pos + term_neg))


def _finish(hist):
    return pl.pallas_call(
        _finish_body,
        out_shape=jax.ShapeDtypeStruct((1, 1), jnp.float32),
    )(hist)


def kernel(logits, labels):
    x = logits.reshape(ROWS, COLS)
    l = labels.reshape(ROWS, COLS)
    mn, mx = _minmax(x, l)
    emin = mn[0, 0]
    span = mx[0, 0] - emin
    scale = jnp.where(span > 0.0, (NB - 1.0) / span, 0.0)
    params = jnp.zeros((16,), jnp.float32)
    params = params.at[0].set(emin).at[1].set(scale)
    hist = _sc_hist(logits.reshape(P_TOTAL), labels.reshape(P_TOTAL), params)
    loss = _finish(hist.reshape(NTILES, 4, 128, 128))
    return loss[0, 0]


# speedup vs baseline: 24.0173x; 1.4181x over previous
import functools

import jax
import jax.numpy as jnp
from jax import lax
from jax.experimental import pallas as pl
from jax.experimental.pallas import tpu as pltpu
from jax.experimental.pallas import tpu_sc as plsc

P_TOTAL = 8 * 1 * 512 * 512
NB = 4096
NTILES = 32
CHUNK = P_TOTAL // NTILES
W = 8192
NWP = CHUNK // W // 2
UNROLL = 8
ROWS = 2048
COLS = P_TOTAL // ROWS
MM_BLK = 256


def _minmax_body(x_ref, l_ref, mn_ref, mx_ref):
    x = x_ref[...]
    lf = l_ref[...].astype(jnp.float32)
    e = 1.0 - x * (2.0 * lf - 1.0)
    m = jnp.min(e)
    M = jnp.max(e)

    @pl.when(pl.program_id(0) == 0)
    def _():
        mn_ref[...] = jnp.full((1, 1), m)
        mx_ref[...] = jnp.full((1, 1), M)

    mn_ref[...] = jnp.minimum(mn_ref[...], m)
    mx_ref[...] = jnp.maximum(mx_ref[...], M)


def _minmax(x2d, l2d):
    return pl.pallas_call(
        _minmax_body,
        grid=(ROWS // MM_BLK,),
        in_specs=[
            pl.BlockSpec((MM_BLK, COLS), lambda i: (i, 0)),
            pl.BlockSpec((MM_BLK, COLS), lambda i: (i, 0)),
        ],
        out_specs=[
            pl.BlockSpec((1, 1), lambda i: (0, 0)),
            pl.BlockSpec((1, 1), lambda i: (0, 0)),
        ],
        out_shape=[
            jax.ShapeDtypeStruct((1, 1), jnp.float32),
            jax.ShapeDtypeStruct((1, 1), jnp.float32),
        ],
    )(x2d, l2d)


def _sc_hist_body(x_hbm, l_hbm, params_hbm, out_hbm,
                  table, xb, lb, pbuf, sx0, sl0, sx1, sl1):
    wid = lax.axis_index("s") * 2 + lax.axis_index("c")
    base = wid * CHUNK

    pltpu.sync_copy(params_hbm, pbuf)
    pv = pbuf[pl.ds(0, 16)]
    emin = pv[0]
    scale = pv[1]
    offs = emin * scale

    pltpu.async_copy(x_hbm.at[pl.ds(base, W)], xb.at[0], sx0)
    pltpu.async_copy(l_hbm.at[pl.ds(base, W)], lb.at[0], sl0)

    zeros16 = jnp.zeros((16,), jnp.float32)

    def zero_body(i, c):
        for u in range(UNROLL):
            table[pl.ds((i * UNROLL + u) * 16, 16)] = zeros16
        return c

    lax.fori_loop(0, 4 * NB // 16 // UNROLL, zero_body, 0)

    ones = jnp.full((16,), 1.0, jnp.float32)

    def process(slot):
        def elem_body(j, c2):
            for u in range(UNROLL):
                off = (j * UNROLL + u) * 16
                x = xb[slot, pl.ds(off, 16)]
                li = lb[slot, pl.ds(off, 16)]
                lf = li.astype(jnp.float32)
                e = 1.0 - x * (2.0 * lf - 1.0)
                g = jnp.where(e > 0.0, e + 1.0, jnp.exp(e))
                b = (e * scale - offs).astype(jnp.int32)
                b = jnp.clip(b, 0, NB - 1)
                idx = b + li * NB
                plsc.addupdate_scatter(table, [idx], ones)
                plsc.addupdate_scatter(table, [idx + 2 * NB], g)
            return c2

        lax.fori_loop(0, W // 16 // UNROLL, elem_body, 0)

    def wait_slot(slot, sx, sl):
        pltpu.make_async_copy(x_hbm.at[pl.ds(0, W)], xb.at[slot], sx).wait()
        pltpu.make_async_copy(l_hbm.at[pl.ds(0, W)], lb.at[slot], sl).wait()

    def wp_body(wp, c):
        w0 = wp * 2
        pltpu.async_copy(x_hbm.at[pl.ds(base + (w0 + 1) * W, W)], xb.at[1], sx1)
        pltpu.async_copy(l_hbm.at[pl.ds(base + (w0 + 1) * W, W)], lb.at[1], sl1)
        wait_slot(0, sx0, sl0)
        process(0)

        @pl.when(wp < NWP - 1)
        def _():
            pltpu.async_copy(x_hbm.at[pl.ds(base + (w0 + 2) * W, W)],
                             xb.at[0], sx0)
            pltpu.async_copy(l_hbm.at[pl.ds(base + (w0 + 2) * W, W)],
                             lb.at[0], sl0)

        wait_slot(1, sx1, sl1)
        process(1)
        return c

    lax.fori_loop(0, NWP, wp_body, 0)
    pltpu.sync_copy(table, out_hbm.at[wid])


_sc_hist = functools.partial(
    pl.kernel,
    out_type=jax.ShapeDtypeStruct((NTILES, 4 * NB), jnp.float32),
    mesh=plsc.VectorSubcoreMesh(core_axis_name="c", subcore_axis_name="s"),
    compiler_params=pltpu.CompilerParams(needs_layout_passes=False),
    scratch_types=[
        pltpu.VMEM((4 * NB,), jnp.float32),
        pltpu.VMEM((2, W), jnp.float32),
        pltpu.VMEM((2, W), jnp.int32),
        pltpu.VMEM((16,), jnp.float32),
        pltpu.SemaphoreType.DMA,
        pltpu.SemaphoreType.DMA,
        pltpu.SemaphoreType.DMA,
        pltpu.SemaphoreType.DMA,
    ],
)(_sc_hist_body)


def _finish_body(h_ref, out_ref):
    h = h_ref[...]
    s = jnp.sum(h, axis=0)
    cp = s[0]
    cn = s[1]
    gp = s[2]
    gn = s[3]

    rows = lax.broadcasted_iota(jnp.int32, (128, 128), 0)
    cols = lax.broadcasted_iota(jnp.int32, (128, 128), 1)
    upper = (rows <= cols).astype(jnp.float32)
    rows32 = lax.broadcasted_iota(jnp.int32, (32, 32), 0)
    cols32 = lax.broadcasted_iota(jnp.int32, (32, 32), 1)
    lstrict = (cols32 < rows32).astype(jnp.float32)

    def incl_cumsum(a):
        within = jnp.dot(a, upper, preferred_element_type=jnp.float32)
        row_tot = jnp.sum(a, axis=1, keepdims=True)
        row_pre = jnp.dot(lstrict, row_tot,
                          preferred_element_type=jnp.float32)
        return within + row_pre

    G = jnp.sum(cp)
    P = G - incl_cumsum(cp)
    N = jnp.sum(cn) - incl_cumsum(cn)
    tn = cn
    tp = cp
    GN = G + N
    term_pos = jnp.where(tp > 0.0, gp / (GN + tn), 0.0)
    grp = jnp.where(GN > 0.0,
                    (G - P) * (1.0 / GN - 1.0 / (GN + tn)),
                    1.0)
    term_neg = jnp.where(tn > 0.0, (gn / tn) * grp, 0.0)
    out_ref[...] = jnp.full((1, 1), jnp.sum(term_pos + term_neg))


def _finish(hist):
    return pl.pallas_call(
        _finish_body,
        out_shape=jax.ShapeDtypeStruct((1, 1), jnp.float32),
    )(hist)


def kernel(logits, labels):
    x = logits.reshape(ROWS, COLS)
    l = labels.reshape(ROWS, COLS)
    mn, mx = _minmax(x, l)
    emin = mn[0, 0]
    span = mx[0, 0] - emin
    scale = jnp.where(span > 0.0, (NB - 1.0) / span, 0.0)
    params = jnp.zeros((16,), jnp.float32)
    params = params.at[0].set(emin).at[1].set(scale)
    hist = _sc_hist(logits.reshape(P_TOTAL), labels.reshape(P_TOTAL), params)
    loss = _finish(hist.reshape(NTILES, 4, 32, 128))
    return loss[0, 0]


# speedup vs baseline: 40.6907x; 1.6942x over previous
import functools

import jax
import jax.numpy as jnp
from jax import lax
from jax.experimental import pallas as pl
from jax.experimental.pallas import tpu as pltpu
from jax.experimental.pallas import tpu_sc as plsc

P_TOTAL = 8 * 1 * 512 * 512
NB = 4096
NTILES = 32
CHUNK = P_TOTAL // NTILES
W = 8192
NWP = CHUNK // W // 2
UNROLL = 8
ROWS = 2048
COLS = P_TOTAL // ROWS
MM_BLK = 256


def _minmax_body(x_ref, l_ref, mn_ref, mx_ref):
    x = x_ref[...]
    lf = l_ref[...].astype(jnp.float32)
    e = 1.0 - x * (2.0 * lf - 1.0)
    m = jnp.min(e)
    M = jnp.max(e)

    @pl.when(pl.program_id(0) == 0)
    def _():
        mn_ref[...] = jnp.full((1, 1), m)
        mx_ref[...] = jnp.full((1, 1), M)

    mn_ref[...] = jnp.minimum(mn_ref[...], m)
    mx_ref[...] = jnp.maximum(mx_ref[...], M)


def _minmax(x2d, l2d):
    return pl.pallas_call(
        _minmax_body,
        grid=(ROWS // MM_BLK,),
        in_specs=[
            pl.BlockSpec((MM_BLK, COLS), lambda i: (i, 0)),
            pl.BlockSpec((MM_BLK, COLS), lambda i: (i, 0)),
        ],
        out_specs=[
            pl.BlockSpec((1, 1), lambda i: (0, 0)),
            pl.BlockSpec((1, 1), lambda i: (0, 0)),
        ],
        out_shape=[
            jax.ShapeDtypeStruct((1, 1), jnp.float32),
            jax.ShapeDtypeStruct((1, 1), jnp.float32),
        ],
    )(x2d, l2d)


def _sc_hist_body(x_hbm, l_hbm, params_hbm, out_hbm,
                  table, xb, lb, pbuf, sx0, sl0, sx1, sl1):
    wid = lax.axis_index("s") * 2 + lax.axis_index("c")
    base = wid * CHUNK

    pltpu.sync_copy(params_hbm, pbuf)
    pv = pbuf[pl.ds(0, 16)]
    emin = pv[0]
    scale = pv[1]
    offs = emin * scale

    pltpu.async_copy(x_hbm.at[pl.ds(base, W)], xb.at[0], sx0)
    pltpu.async_copy(l_hbm.at[pl.ds(base, W)], lb.at[0], sl0)

    zeros16 = jnp.zeros((16,), jnp.float32)

    def zero_body(i, c):
        for u in range(UNROLL):
            table[pl.ds((i * UNROLL + u) * 16, 16)] = zeros16
        return c

    lax.fori_loop(0, 4 * NB // 16 // UNROLL, zero_body, 0)

    ones = jnp.full((16,), 1.0, jnp.float32)

    def process(slot):
        @plsc.parallel_loop(0, W // 16, unroll=UNROLL)
        def _elem_body(j):
            off = j * 16
            x = xb[slot, pl.ds(off, 16)]
            li = lb[slot, pl.ds(off, 16)]
            lf = li.astype(jnp.float32)
            e = 1.0 - x * (2.0 * lf - 1.0)
            g = jnp.where(e > 0.0, e + 1.0, jnp.exp(e))
            b = (e * scale - offs).astype(jnp.int32)
            b = jnp.clip(b, 0, NB - 1)
            idx = b + li * NB
            plsc.addupdate_scatter(table, [idx], ones)
            plsc.addupdate_scatter(table, [idx + 2 * NB], g)

    def wait_slot(slot, sx, sl):
        pltpu.make_async_copy(x_hbm.at[pl.ds(0, W)], xb.at[slot], sx).wait()
        pltpu.make_async_copy(l_hbm.at[pl.ds(0, W)], lb.at[slot], sl).wait()

    def wp_body(wp, c):
        w0 = wp * 2
        pltpu.async_copy(x_hbm.at[pl.ds(base + (w0 + 1) * W, W)], xb.at[1], sx1)
        pltpu.async_copy(l_hbm.at[pl.ds(base + (w0 + 1) * W, W)], lb.at[1], sl1)
        wait_slot(0, sx0, sl0)
        process(0)

        @pl.when(wp < NWP - 1)
        def _():
            pltpu.async_copy(x_hbm.at[pl.ds(base + (w0 + 2) * W, W)],
                             xb.at[0], sx0)
            pltpu.async_copy(l_hbm.at[pl.ds(base + (w0 + 2) * W, W)],
                             lb.at[0], sl0)

        wait_slot(1, sx1, sl1)
        process(1)
        return c

    lax.fori_loop(0, NWP, wp_body, 0)
    pltpu.sync_copy(table, out_hbm.at[wid])


_sc_hist = functools.partial(
    pl.kernel,
    out_type=jax.ShapeDtypeStruct((NTILES, 4 * NB), jnp.float32),
    mesh=plsc.VectorSubcoreMesh(core_axis_name="c", subcore_axis_name="s"),
    compiler_params=pltpu.CompilerParams(needs_layout_passes=False),
    scratch_types=[
        pltpu.VMEM((4 * NB,), jnp.float32),
        pltpu.VMEM((2, W), jnp.float32),
        pltpu.VMEM((2, W), jnp.int32),
        pltpu.VMEM((16,), jnp.float32),
        pltpu.SemaphoreType.DMA,
        pltpu.SemaphoreType.DMA,
        pltpu.SemaphoreType.DMA,
        pltpu.SemaphoreType.DMA,
    ],
)(_sc_hist_body)


def _finish_body(h_ref, out_ref):
    h = h_ref[...]
    s = jnp.sum(h, axis=0)
    cp = s[0]
    cn = s[1]
    gp = s[2]
    gn = s[3]

    rows = lax.broadcasted_iota(jnp.int32, (128, 128), 0)
    cols = lax.broadcasted_iota(jnp.int32, (128, 128), 1)
    upper = (rows <= cols).astype(jnp.float32)
    rows32 = lax.broadcasted_iota(jnp.int32, (32, 32), 0)
    cols32 = lax.broadcasted_iota(jnp.int32, (32, 32), 1)
    lstrict = (cols32 < rows32).astype(jnp.float32)

    def incl_cumsum(a):
        within = jnp.dot(a, upper, preferred_element_type=jnp.float32)
        row_tot = jnp.sum(a, axis=1, keepdims=True)
        row_pre = jnp.dot(lstrict, row_tot,
                          preferred_element_type=jnp.float32)
        return within + row_pre

    G = jnp.sum(cp)
    P = G - incl_cumsum(cp)
    N = jnp.sum(cn) - incl_cumsum(cn)
    tn = cn
    tp = cp
    GN = G + N
    term_pos = jnp.where(tp > 0.0, gp / (GN + tn), 0.0)
    grp = jnp.where(GN > 0.0,
                    (G - P) * (1.0 / GN - 1.0 / (GN + tn)),
                    1.0)
    term_neg = jnp.where(tn > 0.0, (gn / tn) * grp, 0.0)
    out_ref[...] = jnp.full((1, 1), jnp.sum(term_pos + term_neg))


def _finish(hist):
    return pl.pallas_call(
        _finish_body,
        out_shape=jax.ShapeDtypeStruct((1, 1), jnp.float32),
    )(hist)


def kernel(logits, labels):
    x = logits.reshape(ROWS, COLS)
    l = labels.reshape(ROWS, COLS)
    mn, mx = _minmax(x, l)
    emin = mn[0, 0]
    span = mx[0, 0] - emin
    scale = jnp.where(span > 0.0, (NB - 1.0) / span, 0.0)
    params = jnp.zeros((16,), jnp.float32)
    params = params.at[0].set(emin).at[1].set(scale)
    hist = _sc_hist(logits.reshape(P_TOTAL), labels.reshape(P_TOTAL), params)
    loss = _finish(hist.reshape(NTILES, 4, 32, 128))
    return loss[0, 0]


# speedup vs baseline: 47.7735x; 1.1741x over previous
import functools

import jax
import jax.numpy as jnp
from jax import lax
from jax.experimental import pallas as pl
from jax.experimental.pallas import tpu as pltpu
from jax.experimental.pallas import tpu_sc as plsc

P_TOTAL = 8 * 1 * 512 * 512
NB = 4096
NTILES = 32
UNROLL = 8
ROWS = 2048
COLS = P_TOTAL // ROWS
MM_BLK = 256


def _minmax_body(x_ref, l_ref, pk_ref, mn_ref, mx_ref):
    x = x_ref[...]
    l = l_ref[...]
    lf = l.astype(jnp.float32)
    e = 1.0 - x * (2.0 * lf - 1.0)
    pk_ref[...] = (lax.bitcast_convert_type(e, jnp.int32) & -2) | l
    m = jnp.min(e)
    M = jnp.max(e)

    @pl.when(pl.program_id(0) == 0)
    def _():
        mn_ref[...] = jnp.full((1, 1), m)
        mx_ref[...] = jnp.full((1, 1), M)

    mn_ref[...] = jnp.minimum(mn_ref[...], m)
    mx_ref[...] = jnp.maximum(mx_ref[...], M)


def _minmax(x2d, l2d):
    return pl.pallas_call(
        _minmax_body,
        grid=(ROWS // MM_BLK,),
        in_specs=[
            pl.BlockSpec((MM_BLK, COLS), lambda i: (i, 0)),
            pl.BlockSpec((MM_BLK, COLS), lambda i: (i, 0)),
        ],
        out_specs=[
            pl.BlockSpec((MM_BLK, COLS), lambda i: (i, 0)),
            pl.BlockSpec((1, 1), lambda i: (0, 0)),
            pl.BlockSpec((1, 1), lambda i: (0, 0)),
        ],
        out_shape=[
            jax.ShapeDtypeStruct((ROWS, COLS), jnp.int32),
            jax.ShapeDtypeStruct((1, 1), jnp.float32),
            jax.ShapeDtypeStruct((1, 1), jnp.float32),
        ],
    )(x2d, l2d)


WR = 8
TROWS = ROWS // NTILES
NWP = TROWS // WR // 2
VPW = WR * COLS // 16


def _sc_hist_body(pk_hbm, params_hbm, out_hbm, table, xb, pbuf, s0, s1):
    wid = lax.axis_index("s") * 2 + lax.axis_index("c")
    base = wid * TROWS

    pltpu.sync_copy(params_hbm, pbuf)
    pv = pbuf[pl.ds(0, 16)]
    emin = pv[0]
    scale = pv[1]
    offs = emin * scale

    pltpu.async_copy(pk_hbm.at[pl.ds(base, WR)], xb.at[0], s0)

    zeros16 = jnp.zeros((16,), jnp.float32)

    def zero_body(i, c):
        for u in range(UNROLL):
            table[pl.ds((i * UNROLL + u) * 16, 16)] = zeros16
        return c

    lax.fori_loop(0, 4 * NB // 16 // UNROLL, zero_body, 0)

    ones = jnp.full((16,), 1.0, jnp.float32)

    def process(slot):
        @plsc.parallel_loop(0, VPW, unroll=UNROLL)
        def _elem_body(j):
            r = lax.shift_right_logical(j, 6)
            cc = (j & 63) * 16
            pk = xb[slot, r, pl.ds(cc, 16)]
            li = pk & 1
            e = lax.bitcast_convert_type(pk & -2, jnp.float32)
            g = jnp.where(e > 0.0, e + 1.0, jnp.exp(e))
            b = (e * scale - offs).astype(jnp.int32)
            b = jnp.clip(b, 0, NB - 1)
            idx = b + li * NB
            plsc.addupdate_scatter(table, [idx], ones)
            plsc.addupdate_scatter(table, [idx + 2 * NB], g)

    def wait_slot(slot, sem):
        pltpu.make_async_copy(pk_hbm.at[pl.ds(0, WR)], xb.at[slot], sem).wait()

    def wp_body(wp, c):
        w0 = wp * 2
        pltpu.async_copy(pk_hbm.at[pl.ds(base + (w0 + 1) * WR, WR)],
                         xb.at[1], s1)
        wait_slot(0, s0)
        process(0)

        @pl.when(wp < NWP - 1)
        def _():
            pltpu.async_copy(pk_hbm.at[pl.ds(base + (w0 + 2) * WR, WR)],
                             xb.at[0], s0)

        wait_slot(1, s1)
        process(1)
        return c

    lax.fori_loop(0, NWP, wp_body, 0)
    pltpu.sync_copy(table, out_hbm.at[wid])


_sc_hist = functools.partial(
    pl.kernel,
    out_type=jax.ShapeDtypeStruct((NTILES, 4 * NB), jnp.float32),
    mesh=plsc.VectorSubcoreMesh(core_axis_name="c", subcore_axis_name="s"),
    compiler_params=pltpu.CompilerParams(needs_layout_passes=False),
    scratch_types=[
        pltpu.VMEM((4 * NB,), jnp.float32),
        pltpu.VMEM((2, WR, COLS), jnp.int32),
        pltpu.VMEM((16,), jnp.float32),
        pltpu.SemaphoreType.DMA,
        pltpu.SemaphoreType.DMA,
    ],
)(_sc_hist_body)


def _finish_body(h_ref, out_ref):
    h = h_ref[...]
    s = jnp.sum(h, axis=0)
    cp = s[0]
    cn = s[1]
    gp = s[2]
    gn = s[3]

    rows = lax.broadcasted_iota(jnp.int32, (128, 128), 0)
    cols = lax.broadcasted_iota(jnp.int32, (128, 128), 1)
    upper = (rows <= cols).astype(jnp.float32)
    rows32 = lax.broadcasted_iota(jnp.int32, (32, 32), 0)
    cols32 = lax.broadcasted_iota(jnp.int32, (32, 32), 1)
    lstrict = (cols32 < rows32).astype(jnp.float32)

    def incl_cumsum(a):
        within = jnp.dot(a, upper, preferred_element_type=jnp.float32)
        row_tot = jnp.sum(a, axis=1, keepdims=True)
        row_pre = jnp.dot(lstrict, row_tot,
                          preferred_element_type=jnp.float32)
        return within + row_pre

    G = jnp.sum(cp)
    P = G - incl_cumsum(cp)
    N = jnp.sum(cn) - incl_cumsum(cn)
    tn = cn
    tp = cp
    GN = G + N
    term_pos = jnp.where(tp > 0.0, gp / (GN + tn), 0.0)
    grp = jnp.where(GN > 0.0,
                    (G - P) * (1.0 / GN - 1.0 / (GN + tn)),
                    1.0)
    term_neg = jnp.where(tn > 0.0, (gn / tn) * grp, 0.0)
    out_ref[...] = jnp.full((1, 1), jnp.sum(term_pos + term_neg))


def _finish(hist):
    return pl.pallas_call(
        _finish_body,
        out_shape=jax.ShapeDtypeStruct((1, 1), jnp.float32),
    )(hist)


def kernel(logits, labels):
    x = logits.reshape(ROWS, COLS)
    l = labels.reshape(ROWS, COLS)
    pk, mn, mx = _minmax(x, l)
    emin = mn[0, 0]
    span = mx[0, 0] - emin
    scale = jnp.where(span > 0.0, (NB - 1.0) / span, 0.0)
    params = jnp.zeros((16,), jnp.float32)
    params = params.at[0].set(emin).at[1].set(scale)
    hist = _sc_hist(pk, params)
    loss = _finish(hist.reshape(NTILES, 4, 32, 128))
    return loss[0, 0]


# speedup vs baseline: 71.2655x; 1.4917x over previous
import functools

import jax
import jax.numpy as jnp
from jax import lax
from jax.experimental import pallas as pl
from jax.experimental.pallas import tpu as pltpu
from jax.experimental.pallas import tpu_sc as plsc

B0, B1, B2, B3 = 8, 1, 512, 512
P_TOTAL = B0 * B1 * B2 * B3
NB = 4096
NTILES = 32
UNROLL = 8
WR = 16
RPT = B2 // 4
NWP = RPT // WR // 2
VPW = WR * B3 // 16


def _prep_body(x_ref, l_ref, pk_ref, params_ref, mn_scr, mx_scr):
    x = x_ref[...]
    l = l_ref[...]
    lf = l.astype(jnp.float32)
    e = 1.0 - x * (2.0 * lf - 1.0)
    pk_ref[...] = (lax.bitcast_convert_type(e, jnp.int32) & -2) | l
    m = jnp.full((1, 1), jnp.min(e))
    M = jnp.full((1, 1), jnp.max(e))

    @pl.when(pl.program_id(0) == 0)
    def _():
        mn_scr[...] = m
        mx_scr[...] = M

    mn_scr[...] = jnp.minimum(mn_scr[...], m)
    mx_scr[...] = jnp.maximum(mx_scr[...], M)

    @pl.when(pl.program_id(0) == B0 - 1)
    def _():
        emin = mn_scr[...]
        span = mx_scr[...] - emin
        scale = jnp.where(span > 0.0, (NB - 1.0) / span, 0.0)
        lane = lax.broadcasted_iota(jnp.int32, (1, 16), 1)
        pv = jnp.where(lane == 0, emin, jnp.where(lane == 1, scale, 0.0))
        params_ref[...] = pv


def _prep(x4d, l4d):
    return pl.pallas_call(
        _prep_body,
        grid=(B0,),
        in_specs=[
            pl.BlockSpec((1, B1, B2, B3), lambda i: (i, 0, 0, 0)),
            pl.BlockSpec((1, B1, B2, B3), lambda i: (i, 0, 0, 0)),
        ],
        out_specs=[
            pl.BlockSpec((1, B1, B2, B3), lambda i: (i, 0, 0, 0)),
            pl.BlockSpec((1, 16), lambda i: (0, 0)),
        ],
        out_shape=[
            jax.ShapeDtypeStruct((B0, B1, B2, B3), jnp.int32),
            jax.ShapeDtypeStruct((1, 16), jnp.float32),
        ],
        scratch_shapes=[
            pltpu.VMEM((1, 1), jnp.float32),
            pltpu.VMEM((1, 1), jnp.float32),
        ],
    )(x4d, l4d)


def _sc_hist_body(pk_hbm, params_hbm, out_hbm, table, xb, pbuf, s0, s1):
    wid = lax.axis_index("s") * 2 + lax.axis_index("c")
    bidx = wid // 4
    r0 = (wid % 4) * RPT

    pltpu.sync_copy(params_hbm, pbuf)
    pv = pbuf[0, pl.ds(0, 16)]
    emin = pv[0]
    scale = pv[1]
    offs = emin * scale

    def win_src(w):
        return pk_hbm.at[bidx, 0, pl.ds(r0 + w * WR, WR), :]

    pltpu.async_copy(win_src(0), xb.at[0], s0)

    zeros16 = jnp.zeros((16,), jnp.float32)

    def zero_body(i, c):
        for u in range(UNROLL):
            table[pl.ds((i * UNROLL + u) * 16, 16)] = zeros16
        return c

    lax.fori_loop(0, 4 * NB // 16 // UNROLL, zero_body, 0)

    ones = jnp.full((16,), 1.0, jnp.float32)

    def process(slot):
        @plsc.parallel_loop(0, VPW, unroll=UNROLL)
        def _elem_body(j):
            r = lax.shift_right_logical(j, 5)
            cc = (j & 31) * 16
            pk = xb[slot, r, pl.ds(cc, 16)]
            li = pk & 1
            e = lax.bitcast_convert_type(pk & -2, jnp.float32)
            g = jnp.where(e > 0.0, e + 1.0, jnp.exp(e))
            b = (e * scale - offs).astype(jnp.int32)
            b = jnp.clip(b, 0, NB - 1)
            idx = b + li * NB
            plsc.addupdate_scatter(table, [idx], ones)
            plsc.addupdate_scatter(table, [idx + 2 * NB], g)

    def wait_slot(slot, sem):
        pltpu.make_async_copy(win_src(0), xb.at[slot], sem).wait()

    def wp_body(wp, c):
        w0 = wp * 2
        pltpu.async_copy(win_src(w0 + 1), xb.at[1], s1)
        wait_slot(0, s0)
        process(0)

        @pl.when(wp < NWP - 1)
        def _():
            pltpu.async_copy(win_src(w0 + 2), xb.at[0], s0)

        wait_slot(1, s1)
        process(1)
        return c

    lax.fori_loop(0, NWP, wp_body, 0)
    pltpu.sync_copy(table, out_hbm.at[wid])


_sc_hist = functools.partial(
    pl.kernel,
    out_type=jax.ShapeDtypeStruct((NTILES, 4 * NB), jnp.float32),
    mesh=plsc.VectorSubcoreMesh(core_axis_name="c", subcore_axis_name="s"),
    compiler_params=pltpu.CompilerParams(needs_layout_passes=False),
    scratch_types=[
        pltpu.VMEM((4 * NB,), jnp.float32),
        pltpu.VMEM((2, WR, B3), jnp.int32),
        pltpu.VMEM((1, 16), jnp.float32),
        pltpu.SemaphoreType.DMA,
        pltpu.SemaphoreType.DMA,
    ],
)(_sc_hist_body)


def _finish_body(h_ref, out_ref):
    h = h_ref[...]
    s = jnp.sum(h, axis=0)
    s4 = s.reshape(4, 32, 128)
    cp = s4[0]
    cn = s4[1]
    gp = s4[2]
    gn = s4[3]

    rows = lax.broadcasted_iota(jnp.int32, (128, 128), 0)
    cols = lax.broadcasted_iota(jnp.int32, (128, 128), 1)
    upper = (rows <= cols).astype(jnp.float32)
    rows32 = lax.broadcasted_iota(jnp.int32, (32, 32), 0)
    cols32 = lax.broadcasted_iota(jnp.int32, (32, 32), 1)
    lstrict = (cols32 < rows32).astype(jnp.float32)

    def incl_cumsum(a):
        within = jnp.dot(a, upper, preferred_element_type=jnp.float32)
        row_tot = jnp.sum(a, axis=1, keepdims=True)
        row_pre = jnp.dot(lstrict, row_tot,
                          preferred_element_type=jnp.float32)
        return within + row_pre

    G = jnp.sum(cp)
    P = G - incl_cumsum(cp)
    N = jnp.sum(cn) - incl_cumsum(cn)
    tn = cn
    GN = G + N
    term_pos = jnp.where(gp > 0.0, gp / (GN + tn), 0.0)
    grp = jnp.where(GN > 0.0,
                    (G - P) * (1.0 / GN - 1.0 / (GN + tn)),
                    1.0)
    term_neg = jnp.where(tn > 0.0, (gn / tn) * grp, 0.0)
    out_ref[...] = jnp.full((1, 1), jnp.sum(term_pos + term_neg))


def _finish(hist):
    return pl.pallas_call(
        _finish_body,
        out_shape=jax.ShapeDtypeStruct((1, 1), jnp.float32),
    )(hist)


def kernel(logits, labels):
    pk, params = _prep(logits, labels)
    hist = _sc_hist(pk, params)
    loss = _finish(hist)
    return loss[0, 0]
